# Initial kernel scaffold; baseline (speedup 1.0000x reference)
import jax
import jax.numpy as jnp
from jax.experimental import pallas as pl


def kernel(vote_xyz_center, vote_xyz_corner, vox_pred1, vox_pred2, z_off0, z_off1, x_angle, x_res, x_off0, x_off1, y_angle, y_res, y_off0, y_off1, gt_bboxes, pert_bboxes, num_instance):
    raise NotImplementedError("write your pallas kernel here")


# speedup vs baseline: 15.1322x; 15.1322x over previous
import functools

import jax
import jax.numpy as jnp
import numpy as np
from jax import lax
from jax.experimental import pallas as pl
from jax.experimental.pallas import tpu as pltpu
from jax.experimental.pallas import tpu_sc as plsc

F32 = jnp.float32
I32 = jnp.int32

_XMIN, _XMAX = -3.84, 3.84
_ZMIN = -0.2
_NXG = 77
_NZG = 29
_NAG = 13
_EPS = 1e-4
_XHI = float(np.float32(_NXG - 1 - _EPS))
_ZHI = float(np.float32(_NZG - 1 - _EPS))
_AHI = float(np.float32(_NAG - 1 - _EPS))
_INV_VS = 10.0

_N = 20000
_NP = 20224
_KB = 64
_PLANE = _NXG * _NZG
_SLABA = 87168
_TRI_CH = 2528
_BILF = 1008
_LINF = 80
_BCH = ((0, 320), (320, 320), (640, 320), (960, 304))


def _cues(bbox):
    c = bbox[:, 0:3]
    l = bbox[:, 3]; w = bbox[:, 4]; h = bbox[:, 5]; th = bbox[:, 6]
    ct = jnp.cos(th); st = jnp.sin(th)
    sx = jnp.array([1, 1, 1, 1, -1, -1, -1, -1], F32)
    sy = jnp.array([1, 1, -1, -1, 1, 1, -1, -1], F32)
    sz = jnp.array([1, -1, 1, -1, 1, -1, 1, -1], F32)
    ox = sx[None, :] * (l / 2)[:, None] * ct[:, None] - sy[None, :] * (w / 2)[:, None] * st[:, None]
    oy = sx[None, :] * (l / 2)[:, None] * st[:, None] + sy[None, :] * (w / 2)[:, None] * ct[:, None]
    oz = sz[None, :] * (h / 2)[:, None]
    corners = c[:, None, :] + jnp.stack([ox, oy, oz], axis=2)
    ang = jnp.mod(th, jnp.pi) / (jnp.pi / 12.0)
    dx = c[:, 0] * ct + c[:, 1] * st
    dy = -c[:, 0] * st + c[:, 1] * ct
    clip = lambda v: jnp.clip(v, _XMIN, _XMAX)
    return (c, corners.reshape(-1, 3), ang,
            clip(dx - l / 2), clip(dx + l / 2), clip(dy - w / 2), clip(dy + w / 2),
            c[:, 2] - h / 2, c[:, 2] + h / 2)


def _pad_to(a, n):
    return jnp.concatenate([a, jnp.zeros((n - a.shape[0],), a.dtype)])


def _sc_body(vc, vk, xang, yang, sca, cenp, corp, bilp, linp, vox1, vox2,
             out, slab, vtb, xab, yab, scb, cenb, corb, bilb, linb,
             fbil, flin, vidx, vgb, acc, sem):
    cidx = lax.axis_index("c")
    sidx = lax.axis_index("s")
    slab_id = sidx % 2
    grp = sidx // 2
    row = sidx * 2 + cidx
    lo = slab_id * 38
    lane = lax.iota(I32, 16)
    lane_eq = [lane == j for j in range(16)]
    zero16 = jnp.zeros((16,), F32)
    one16 = jnp.ones((16,), F32)

    def _grid3(px, py, pz):
        x = jnp.minimum(jnp.maximum((px - _XMIN) * _INV_VS, 0.0), _XHI)
        y = jnp.minimum(jnp.maximum((py - _XMIN) * _INV_VS, 0.0), _XHI)
        z = jnp.minimum(jnp.maximum((pz - _ZMIN) * _INV_VS, 0.0), _ZHI)
        x0 = x.astype(I32); y0 = y.astype(I32); z0 = z.astype(I32)
        return x0, y0, z0, x - x0.astype(F32), y - y0.astype(F32), z - z0.astype(F32)

    def _tri_parts(x0, y0, z0, fx, fy, fz, cf, base_x):
        b = (x0 - base_x) * _PLANE + y0 * _NZG + z0
        idxs = (b, b + 1, b + _NZG, b + _NZG + 1,
                b + _PLANE, b + _PLANE + 1, b + _PLANE + _NZG, b + _PLANE + _NZG + 1)
        ax0 = (1.0 - fx) * cf; ax1 = fx * cf
        gy = 1.0 - fy; gz = 1.0 - fz
        w00 = ax0 * gy; w01 = ax0 * fy; w10 = ax1 * gy; w11 = ax1 * fy
        ws = (w00 * gz, w00 * fz, w01 * gz, w01 * fz,
              w10 * gz, w10 * fz, w11 * gz, w11 * fz)
        return idxs, ws

    def _zb(i, _):
        for t in range(8):
            slab[pl.ds(i * 128 + t * 16, 16)] = zero16
        return 0
    lax.fori_loop(0, _SLABA // 128, _zb, 0)

    def _zf(i, _):
        fbil[pl.ds(i * 16, 16)] = zero16
        return 0
    lax.fori_loop(0, 4 * _BILF // 16, _zf, 0)

    def _zl(i, _):
        flin[pl.ds(i * 16, 16)] = zero16
        return 0
    lax.fori_loop(0, 2 * _LINF // 16, _zl, 0)
    acc[...] = zero16

    pltpu.sync_copy(cenp, cenb)
    pltpu.sync_copy(bilp, bilb)
    pltpu.sync_copy(linp, linb)

    @pl.when(cidx == 1)
    def _():
        pltpu.sync_copy(corp, corb)
        for j in range(3):
            pltpu.sync_copy(vk.at[pl.ds(j * _NP + grp * _TRI_CH, _TRI_CH)],
                            vtb.at[pl.ds(j * _TRI_CH, _TRI_CH)])

    @pl.when(cidx == 0)
    def _():
        for j in range(3):
            pltpu.sync_copy(vc.at[pl.ds(j * _NP + grp * _TRI_CH, _TRI_CH)],
                            vtb.at[pl.ds(j * _TRI_CH, _TRI_CH)])

    def _tri_build(pref, npts, ngroups):
        def body(i, _):
            o = i * 16
            x0, y0, z0, fx, fy, fz = _grid3(pref[pl.ds(o, 16)],
                                            pref[pl.ds(npts + o, 16)],
                                            pref[pl.ds(2 * npts + o, 16)])
            cf = pref[pl.ds(3 * npts + o, 16)]
            m0 = (x0 >= lo) & (x0 <= lo + 38)
            m1 = (x0 + 1 >= lo) & (x0 + 1 <= lo + 38)
            yz = y0 * _NZG + z0
            p0 = jnp.minimum(jnp.maximum(x0 - lo, 0), 38)
            p1 = jnp.minimum(jnp.maximum(x0 + 1 - lo, 0), 38)
            b0 = p0 * _PLANE + yz
            b1 = p1 * _PLANE + yz
            idxs = (b0, b0 + 1, b0 + _NZG, b0 + _NZG + 1,
                    b1, b1 + 1, b1 + _NZG, b1 + _NZG + 1)
            _, ws = _tri_parts(x0, y0, z0, fx, fy, fz, cf, lo)
            for j in range(16):
                lm0 = lane_eq[j] & m0
                lm1 = lane_eq[j] & m1
                for t in range(8):
                    plsc.addupdate_scatter(slab, [idxs[t]], ws[t],
                                           mask=lm1 if t >= 4 else lm0)
            return 0
        lax.fori_loop(0, ngroups, body, 0)

    @pl.when(cidx == 0)
    def _():
        _tri_build(cenb, 128, 8)

    @pl.when(cidx == 1)
    def _():
        _tri_build(corb, 1024, 64)

    def _vox_taps(pref, npts, gstart, ngroups, src):
        def body(i, _):
            o = (gstart + i) * 16
            x0, y0, z0, fx, fy, fz = _grid3(pref[pl.ds(o, 16)],
                                            pref[pl.ds(npts + o, 16)],
                                            pref[pl.ds(2 * npts + o, 16)])
            cf = pref[pl.ds(3 * npts + o, 16)]
            idxs, ws = _tri_parts(x0, y0, z0, fx, fy, fz, cf, 0)
            for t in range(8):
                vidx[pl.ds(t * 16, 16)] = idxs[t]
            pltpu.async_copy(src.at[vidx], vgb, sem).wait()
            sacc = zero16
            for t in range(8):
                sacc = sacc + vgb[pl.ds(t * 16, 16)] * ws[t]
            acc[...] = acc[...] + sacc
            return 0
        lax.fori_loop(0, ngroups, body, 0)

    @pl.when(cidx == 1)
    def _():
        _vox_taps(corb, 1024, sidx * 4, 4, vox2)

    @pl.when((cidx == 1) & (sidx == 0))
    def _():
        _vox_taps(cenb, 128, 0, 8, vox1)

    @pl.when(cidx == 0)
    def _():
        def body(i, _):
            o = i * 16
            a = jnp.minimum(jnp.maximum(bilb[pl.ds(o, 16)], 0.0), _AHI)
            cf = bilb[pl.ds(5 * 128 + o, 16)]
            a0 = a.astype(I32)
            fa = a - a0.astype(F32)
            wa0 = (1.0 - fa) * cf; wa1 = fa * cf
            for f in range(4):
                yv = bilb[pl.ds((1 + f) * 128 + o, 16)]
                y = jnp.minimum(jnp.maximum((yv - _XMIN) * _INV_VS, 0.0), _XHI)
                y0 = y.astype(I32)
                fy = y - y0.astype(F32)
                b = f * _BILF + a0 * _NXG + y0
                idxs = (b, b + 1, b + _NXG, b + _NXG + 1)
                ws = (wa0 * (1.0 - fy), wa0 * fy, wa1 * (1.0 - fy), wa1 * fy)
                for j in range(16):
                    for t in range(4):
                        plsc.addupdate_scatter(fbil, [idxs[t]], ws[t], mask=lane_eq[j])
            cfl = linb[pl.ds(2 * 128 + o, 16)]
            for f in range(2):
                zv = linb[pl.ds(f * 128 + o, 16)]
                z = jnp.minimum(jnp.maximum((zv - _XMIN) * _INV_VS, 0.0), _XHI)
                z0 = z.astype(I32)
                fz = z - z0.astype(F32)
                b = f * _LINF + z0
                w0 = (1.0 - fz) * cfl; w1 = fz * cfl
                for j in range(16):
                    plsc.addupdate_scatter(flin, [b], w0, mask=lane_eq[j])
                    plsc.addupdate_scatter(flin, [b + 1], w1, mask=lane_eq[j])
            return 0
        lax.fori_loop(0, 8, body, 0)

    base_pt = grp * _TRI_CH

    def _tg(i, _):
        o = i * 16
        x0, y0, z0, fx, fy, fz = _grid3(vtb[pl.ds(o, 16)],
                                        vtb[pl.ds(_TRI_CH + o, 16)],
                                        vtb[pl.ds(2 * _TRI_CH + o, 16)])
        pm = (base_pt + o + lane) < _N
        mm = (x0 >= lo) & (x0 < lo + 38) & pm
        xb = jnp.where(mm, x0, lo)
        idxs, ws = _tri_parts(xb, y0, z0, fx, fy, fz, one16, lo)
        sacc = zero16
        for t in range(8):
            sacc = sacc + plsc.load_gather(slab, [idxs[t]]) * ws[t]
        acc[...] = acc[...] + jnp.where(mm, sacc, 0.0)
        return 0
    lax.fori_loop(0, _TRI_CH // 16, _tg, 0)

    @pl.when(cidx == 0)
    def _():
        for (off, csz) in _BCH:
            cbase = sidx * 1264 + off
            for j in range(12):
                pltpu.sync_copy(xang.at[pl.ds(j * _NP + cbase, csz)],
                                xab.at[pl.ds(j * 320, csz)])
                pltpu.sync_copy(yang.at[pl.ds(j * _NP + cbase, csz)],
                                yab.at[pl.ds(j * 320, csz)])
            for j in range(8):
                pltpu.sync_copy(sca.at[pl.ds(j * _NP + cbase, csz)],
                                scb.at[pl.ds(j * 320, csz)])

            def body(i, _):
                o = i * 16

                def argmax12(ref):
                    best = ref[pl.ds(o, 16)]
                    bidx = zero16
                    for j in range(1, 12):
                        vj = ref[pl.ds(j * 320 + o, 16)]
                        gtm = vj > best
                        best = jnp.where(gtm, vj, best)
                        bidx = jnp.where(gtm, jnp.full((16,), float(j), F32), bidx)
                    return bidx

                pm = (cbase + o + lane) < _N
                total = zero16
                for (aref, rrow, orow0, f0) in ((xab, 0, 1, 0), (yab, 3, 4, 2)):
                    av = argmax12(aref) + scb[pl.ds(rrow * 320 + o, 16)]
                    a = jnp.minimum(jnp.maximum(av, 0.0), _AHI)
                    a0 = a.astype(I32)
                    fa = a - a0.astype(F32)
                    for f in range(2):
                        yv = scb[pl.ds((orow0 + f) * 320 + o, 16)]
                        y = jnp.minimum(jnp.maximum((yv - _XMIN) * _INV_VS, 0.0), _XHI)
                        y0 = y.astype(I32)
                        fy = y - y0.astype(F32)
                        b = (f0 + f) * _BILF + a0 * _NXG + y0
                        g00 = plsc.load_gather(fbil, [b])
                        g01 = plsc.load_gather(fbil, [b + 1])
                        g10 = plsc.load_gather(fbil, [b + _NXG])
                        g11 = plsc.load_gather(fbil, [b + _NXG + 1])
                        total = total + ((1.0 - fa) * (g00 * (1.0 - fy) + g01 * fy)
                                         + fa * (g10 * (1.0 - fy) + g11 * fy))
                for f in range(2):
                    zv = scb[pl.ds((6 + f) * 320 + o, 16)]
                    z = jnp.minimum(jnp.maximum((zv - _XMIN) * _INV_VS, 0.0), _XHI)
                    z0 = z.astype(I32)
                    fz = z - z0.astype(F32)
                    b = f * _LINF + z0
                    total = total + (plsc.load_gather(flin, [b]) * (1.0 - fz)
                                     + plsc.load_gather(flin, [b + 1]) * fz)
                acc[...] = acc[...] + jnp.where(pm, total, 0.0)
                return 0
            lax.fori_loop(0, csz // 16, body, 0)

    pltpu.sync_copy(acc, out.at[pl.ds(row * 16, 16)])


@functools.partial(
    pl.kernel,
    out_type=jax.ShapeDtypeStruct((512,), F32),
    mesh=plsc.VectorSubcoreMesh(core_axis_name="c", subcore_axis_name="s"),
    compiler_params=pltpu.CompilerParams(needs_layout_passes=False),
    scratch_types=[
        pltpu.VMEM((_SLABA,), F32),
        pltpu.VMEM((3 * _TRI_CH,), F32),
        pltpu.VMEM((12 * 320,), F32),
        pltpu.VMEM((12 * 320,), F32),
        pltpu.VMEM((8 * 320,), F32),
        pltpu.VMEM((4 * 128,), F32),
        pltpu.VMEM((4 * 1024,), F32),
        pltpu.VMEM((6 * 128,), F32),
        pltpu.VMEM((3 * 128,), F32),
        pltpu.VMEM((4 * _BILF,), F32),
        pltpu.VMEM((2 * _LINF,), F32),
        pltpu.VMEM((128,), I32),
        pltpu.VMEM((128,), F32),
        pltpu.VMEM((16,), F32),
        pltpu.SemaphoreType.DMA,
    ],
)
def _sc_loss(*refs):
    _sc_body(*refs)


def kernel(vote_xyz_center, vote_xyz_corner, vox_pred1, vox_pred2, z_off0, z_off1,
           x_angle, x_res, x_off0, x_off1, y_angle, y_res, y_off0, y_off1,
           gt_bboxes, pert_bboxes, num_instance):
    pad2 = lambda a: jnp.concatenate([a, jnp.zeros((a.shape[0], _NP - _N), F32)], axis=1)
    vc = pad2(vote_xyz_center.reshape(-1, 3).T).reshape(-1)
    vk = pad2(vote_xyz_corner.reshape(-1, 3).T).reshape(-1)
    xang = pad2(x_angle.T).reshape(-1)
    yang = pad2(y_angle.T).reshape(-1)
    sca = jnp.concatenate([_pad_to(x_res, _NP), _pad_to(x_off0, _NP), _pad_to(x_off1, _NP),
                           _pad_to(y_res, _NP), _pad_to(y_off0, _NP), _pad_to(y_off1, _NP),
                           _pad_to(z_off0.reshape(-1), _NP), _pad_to(z_off1.reshape(-1), _NP)])

    cg = _cues(gt_bboxes.reshape(-1, 7))
    cp = _cues(pert_bboxes.reshape(-1, 7))
    m = (jnp.arange(_KB) < num_instance).astype(F32)
    coef_m = jnp.concatenate([m, -m])
    coef_u = jnp.concatenate([jnp.ones((8 * _KB,), F32), -jnp.ones((8 * _KB,), F32)])
    cen = jnp.concatenate([cg[0], cp[0]], axis=0)
    cor = jnp.concatenate([cg[1], cp[1]], axis=0)
    cenp = jnp.concatenate([cen.T.reshape(-1), coef_m])
    corp = jnp.concatenate([cor.T.reshape(-1), coef_u])
    ang = jnp.concatenate([cg[2], cp[2]])
    bilp = jnp.concatenate([ang,
                            jnp.concatenate([cg[3], cp[3]]), jnp.concatenate([cg[4], cp[4]]),
                            jnp.concatenate([cg[5], cp[5]]), jnp.concatenate([cg[6], cp[6]]),
                            coef_m])
    linp = jnp.concatenate([jnp.concatenate([cg[7], cp[7]]),
                            jnp.concatenate([cg[8], cp[8]]), coef_m])

    vox1 = vox_pred1.reshape(-1)
    vox2 = vox_pred2.reshape(-1)

    out = _sc_loss(vc, vk, xang, yang, sca, cenp, corp, bilp, linp, vox1, vox2)
    return jnp.sum(out)
